# baseline (speedup 1.0000x reference)
import functools

import jax
import jax.numpy as jnp
from jax import lax
from jax.experimental import pallas as pl
from jax.experimental.pallas import tpu as pltpu
from jax.experimental.pallas import tpu_sc as plsc

BATCH = 16384
HIST_LEN = 200
EMBED_DIM = 32

NUM_WORKERS = 32
N_TILES = (BATCH // 128) * (HIST_LEN // 8)
TILES_PER_W = N_TILES // NUM_WORKERS
PAD = 129


def _gather_kernel(x5_hbm, table_hbm, out_hbm,
                   idx0, idx1, rows0, rows1, stage,
                   sem_i0, sem_i1, sem_g0, sem_g1, sem_s):
    wid = lax.axis_index("s") * 2 + lax.axis_index("c")
    t_base = wid * TILES_PER_W
    last_t = N_TILES - 1

    e16 = jax.lax.iota(jnp.int32, 16)

    def start_idx(t, idx_v, sem):
        tc = jnp.minimum(t, last_t)
        pltpu.make_async_copy(
            x5_hbm.at[tc // 128, lax.rem(tc, 128)], idx_v, sem).start()

    def wait_idx(idx_v, sem):
        pltpu.make_async_copy(x5_hbm.at[0, 0], idx_v, sem).wait()

    def transpose_store(t, rows_v):
        hh = t // 128
        bh = lax.rem(t, 128)
        for hl in range(8):
            row_lo = hl * 32 + e16
            row_hi = row_lo + 16
            rbase = hl * 128

            def bl_body(bl, col):
                v0 = rows_v[rbase + bl, pl.ds(0, 16)]
                v1 = rows_v[rbase + bl, pl.ds(16, 16)]
                plsc.store_scatter(stage, [row_lo, col], v0)
                plsc.store_scatter(stage, [row_hi, col], v1)
                return col + 1

            lax.fori_loop(0, 128, bl_body, jnp.zeros((16,), jnp.int32),
                          unroll=8)

            for eh in range(4):
                pltpu.make_async_copy(
                    stage.at[pl.ds(hl * 32 + eh * 8, 8), pl.ds(0, 128)],
                    out_hbm.at[hh * 8 + hl, eh, bh],
                    sem_s,
                ).start()

    def drain_stores():
        for _ in range(32):
            pltpu.make_async_copy(
                stage.at[pl.ds(0, 8), pl.ds(0, 128)],
                out_hbm.at[0, 0, 0],
                sem_s,
            ).wait()

    start_idx(t_base, idx0, sem_i0)
    start_idx(t_base + 1, idx1, sem_i1)
    wait_idx(idx0, sem_i0)
    pltpu.make_async_copy(table_hbm.at[idx0], rows0, sem_g0).start()

    def body(i, carry):
        t = t_base + i
        par = lax.rem(i, 2)

        def one(idx_v, rows_v, sem_i, sem_g, idx_n, rows_n, sem_in, sem_gn):
            pltpu.make_async_copy(table_hbm.at[idx_v], rows_v, sem_g).wait()
            start_idx(t + 2, idx_v, sem_i)
            wait_idx(idx_n, sem_in)
            pltpu.make_async_copy(table_hbm.at[idx_n], rows_n, sem_gn).start()
            @pl.when(i > 0)
            def _():
                drain_stores()
            transpose_store(t, rows_v)

        @pl.when(par == 0)
        def _():
            one(idx0, rows0, sem_i0, sem_g0, idx1, rows1, sem_i1, sem_g1)

        @pl.when(par == 1)
        def _():
            one(idx1, rows1, sem_i1, sem_g1, idx0, rows0, sem_i0, sem_g0)

        return carry

    lax.fori_loop(0, TILES_PER_W, body, 0)

    drain_stores()
    if TILES_PER_W % 2 == 0:
        pltpu.make_async_copy(table_hbm.at[idx0], rows0, sem_g0).wait()
        wait_idx(idx1, sem_i1)
    else:
        pltpu.make_async_copy(table_hbm.at[idx1], rows1, sem_g1).wait()
        wait_idx(idx0, sem_i0)


def _table_kernel(tt_hbm, tail_hbm, tlin_hbm, slab0, slab1, trans0, trans1,
                  sem_r0, sem_r1, sem_w0, sem_w1):
    wid = lax.axis_index("s") * 2 + lax.axis_index("c")
    n_full = 1953
    n_s = jnp.where(wid < 1, 62, 61)
    e16 = jax.lax.iota(jnp.int32, 16)

    def slab_of(k):
        return jnp.minimum(wid + 32 * k, n_full - 1)

    def start_read(k, slab, sem):
        s = slab_of(k)
        pltpu.make_async_copy(
            tt_hbm.at[pl.ds(0, 32), pl.ds(s * 512, 512)],
            slab.at[pl.ds(0, 32), pl.ds(0, 512)], sem).start()

    def wait_read(slab, sem):
        pltpu.make_async_copy(
            tt_hbm.at[pl.ds(0, 32), pl.ds(0, 512)],
            slab.at[pl.ds(0, 32), pl.ds(0, 512)], sem).wait()

    def wait_write(trans, sem):
        pltpu.make_async_copy(trans, tlin_hbm.at[pl.ds(0, 128)], sem).wait()

    def transpose(slab, trans, width):
        def jg_body(jg, col):
            for q in range(4):
                v0 = plsc.load_gather(slab, [e16, col])
                v1 = plsc.load_gather(slab, [e16 + 16, col])
                trans[jg, pl.ds(q * 32, 16)] = v0
                trans[jg, pl.ds(q * 32 + 16, 16)] = v1
                col = col + 1
            return col

        lax.fori_loop(0, width // 4, jg_body, jnp.zeros((16,), jnp.int32),
                      unroll=4)

    start_read(0, slab0, sem_r0)

    def body(k, carry):
        par = lax.rem(k, 2)

        def one(slab, sem_r, trans, sem_w, slab_n, sem_rn):
            wait_read(slab, sem_r)
            start_read(k + 1, slab_n, sem_rn)
            @pl.when(k > 1)
            def _():
                wait_write(trans, sem_w)
            transpose(slab, trans, 512)
            s = slab_of(k)
            pltpu.make_async_copy(
                trans, tlin_hbm.at[pl.ds(s * 128, 128)], sem_w).start()

        @pl.when(par == 0)
        def _():
            one(slab0, sem_r0, trans0, sem_w0, slab1, sem_r1)

        @pl.when(par == 1)
        def _():
            one(slab1, sem_r1, trans1, sem_w1, slab0, sem_r0)

        return carry

    lax.fori_loop(0, n_s, body, 0)

    wait_write(trans0, sem_w0)
    wait_write(trans1, sem_w1)

    @pl.when(lax.rem(n_s, 2) == 0)
    def _():
        wait_read(slab0, sem_r0)

    @pl.when(lax.rem(n_s, 2) == 1)
    def _():
        wait_read(slab1, sem_r1)

    @pl.when(wid == 31)
    def _():
        pltpu.sync_copy(tail_hbm, trans0.at[pl.ds(0, 16)])
        pltpu.sync_copy(trans0.at[pl.ds(0, 16)],
                        tlin_hbm.at[pl.ds(n_full * 128, 16)])


@jax.jit
def _table_relayout(tt, tail):
    mesh = plsc.VectorSubcoreMesh(core_axis_name="c", subcore_axis_name="s")
    return pl.kernel(
        _table_kernel,
        mesh=mesh,
        out_type=jax.ShapeDtypeStruct((250000, 128), jnp.float32),
        scratch_types=[
            pltpu.VMEM((32, 513), jnp.float32),
            pltpu.VMEM((32, 513), jnp.float32),
            pltpu.VMEM((128, 128), jnp.float32),
            pltpu.VMEM((128, 128), jnp.float32),
            pltpu.SemaphoreType.DMA,
            pltpu.SemaphoreType.DMA,
            pltpu.SemaphoreType.DMA,
            pltpu.SemaphoreType.DMA,
        ],
        compiler_params=pltpu.CompilerParams(
            use_tc_tiling_on_sc=True, needs_layout_passes=False),
    )(tt, tail)


@jax.jit
def _embedding_lookup(x5, table):
    mesh = plsc.VectorSubcoreMesh(core_axis_name="c", subcore_axis_name="s")
    return pl.kernel(
        _gather_kernel,
        mesh=mesh,
        out_type=jax.ShapeDtypeStruct((HIST_LEN, 4, 128, 8, 128), jnp.float32),
        scratch_types=[
            pltpu.VMEM((1024,), jnp.int32),
            pltpu.VMEM((1024,), jnp.int32),
            pltpu.VMEM((1024, EMBED_DIM), jnp.float32),
            pltpu.VMEM((1024, EMBED_DIM), jnp.float32),
            pltpu.VMEM((256, PAD), jnp.float32),
            pltpu.SemaphoreType.DMA,
            pltpu.SemaphoreType.DMA,
            pltpu.SemaphoreType.DMA,
            pltpu.SemaphoreType.DMA,
            pltpu.SemaphoreType.DMA,
        ],
        compiler_params=pltpu.CompilerParams(
            use_tc_tiling_on_sc=False, needs_layout_passes=False),
    )(x5, table)


def kernel(x, table):
    x5 = (x.T.reshape(25, 8, 128, 128).transpose(0, 2, 1, 3)
          .reshape(25, 128, 1024).astype(jnp.int32))
    tail = table[999936:].reshape(16, 128)
    tlin = _table_relayout(table.T, tail)
    table = tlin.reshape(1000000, EMBED_DIM)
    out5 = _embedding_lookup(x5, table)
    return (out5.transpose(2, 4, 0, 1, 3)
            .reshape(BATCH, HIST_LEN, EMBED_DIM))


# speedup vs baseline: 1.1912x; 1.1912x over previous
import functools

import jax
import jax.numpy as jnp
from jax import lax
from jax.experimental import pallas as pl
from jax.experimental.pallas import tpu as pltpu
from jax.experimental.pallas import tpu_sc as plsc

BATCH = 16384
HIST_LEN = 200
EMBED_DIM = 32

NUM_WORKERS = 32
N_TILES = (BATCH // 128) * (HIST_LEN // 8)
TILES_PER_W = N_TILES // NUM_WORKERS
PAD = 129


def _gather_kernel(x5_hbm, table_hbm, out_hbm,
                   idx0, idx1, rows0, rows1, stage,
                   sem_i0, sem_i1, sem_g0, sem_g1, sem_s):
    wid = lax.axis_index("s") * 2 + lax.axis_index("c")
    t_base = wid * TILES_PER_W
    last_t = N_TILES - 1

    e16 = jax.lax.iota(jnp.int32, 16)

    def start_idx(t, idx_v, sem):
        tc = jnp.minimum(t, last_t)
        pltpu.make_async_copy(
            x5_hbm.at[tc // 128, lax.rem(tc, 128)], idx_v, sem).start()

    def wait_idx(idx_v, sem):
        pltpu.make_async_copy(x5_hbm.at[0, 0], idx_v, sem).wait()

    def transpose_store(t, rows_v):
        hh = t // 128
        bh = lax.rem(t, 128)
        for hl in range(8):
            row_lo = hl * 32 + e16
            row_hi = row_lo + 16
            rbase = hl * 128

            def bl_body(bl, col):
                v0 = rows_v[rbase + bl, pl.ds(0, 16)]
                v1 = rows_v[rbase + bl, pl.ds(16, 16)]
                plsc.store_scatter(stage, [row_lo, col], v0)
                plsc.store_scatter(stage, [row_hi, col], v1)
                return col + 1

            lax.fori_loop(0, 128, bl_body, jnp.zeros((16,), jnp.int32),
                          unroll=8)

            for eh in range(4):
                pltpu.make_async_copy(
                    stage.at[pl.ds(hl * 32 + eh * 8, 8), pl.ds(0, 128)],
                    out_hbm.at[hh * 8 + hl, eh, bh],
                    sem_s,
                ).start()

    def drain_stores():
        for _ in range(32):
            pltpu.make_async_copy(
                stage.at[pl.ds(0, 8), pl.ds(0, 128)],
                out_hbm.at[0, 0, 0],
                sem_s,
            ).wait()

    start_idx(t_base, idx0, sem_i0)
    start_idx(t_base + 1, idx1, sem_i1)
    wait_idx(idx0, sem_i0)
    pltpu.make_async_copy(table_hbm.at[idx0], rows0, sem_g0).start()

    def body(i, carry):
        t = t_base + i
        par = lax.rem(i, 2)

        def one(idx_v, rows_v, sem_i, sem_g, idx_n, rows_n, sem_in, sem_gn):
            pltpu.make_async_copy(table_hbm.at[idx_v], rows_v, sem_g).wait()
            start_idx(t + 2, idx_v, sem_i)
            wait_idx(idx_n, sem_in)
            pltpu.make_async_copy(table_hbm.at[idx_n], rows_n, sem_gn).start()
            @pl.when(i > 0)
            def _():
                drain_stores()
            transpose_store(t, rows_v)

        @pl.when(par == 0)
        def _():
            one(idx0, rows0, sem_i0, sem_g0, idx1, rows1, sem_i1, sem_g1)

        @pl.when(par == 1)
        def _():
            one(idx1, rows1, sem_i1, sem_g1, idx0, rows0, sem_i0, sem_g0)

        return carry

    lax.fori_loop(0, TILES_PER_W, body, 0)

    drain_stores()
    if TILES_PER_W % 2 == 0:
        pltpu.make_async_copy(table_hbm.at[idx0], rows0, sem_g0).wait()
        wait_idx(idx1, sem_i1)
    else:
        pltpu.make_async_copy(table_hbm.at[idx1], rows1, sem_g1).wait()
        wait_idx(idx0, sem_i0)


@jax.jit
def _embedding_lookup(x5, table):
    mesh = plsc.VectorSubcoreMesh(core_axis_name="c", subcore_axis_name="s")
    return pl.kernel(
        _gather_kernel,
        mesh=mesh,
        out_type=jax.ShapeDtypeStruct((HIST_LEN, 4, 128, 8, 128), jnp.float32),
        scratch_types=[
            pltpu.VMEM((1024,), jnp.int32),
            pltpu.VMEM((1024,), jnp.int32),
            pltpu.VMEM((1024, EMBED_DIM), jnp.float32),
            pltpu.VMEM((1024, EMBED_DIM), jnp.float32),
            pltpu.VMEM((256, PAD), jnp.float32),
            pltpu.SemaphoreType.DMA,
            pltpu.SemaphoreType.DMA,
            pltpu.SemaphoreType.DMA,
            pltpu.SemaphoreType.DMA,
            pltpu.SemaphoreType.DMA,
        ],
        compiler_params=pltpu.CompilerParams(
            use_tc_tiling_on_sc=False, needs_layout_passes=False),
    )(x5, table)


def kernel(x, table):
    x5 = (x.T.reshape(25, 8, 128, 128).transpose(0, 2, 1, 3)
          .reshape(25, 128, 1024).astype(jnp.int32))
    out5 = _embedding_lookup(x5, table)
    return (out5.transpose(2, 4, 0, 1, 3)
            .reshape(BATCH, HIST_LEN, EMBED_DIM))
